# baseline (speedup 1.0000x reference)
import functools

import jax
import jax.numpy as jnp
from jax import lax
from jax.experimental import pallas as pl
from jax.experimental.pallas import tpu as pltpu
from jax.experimental.pallas import tpu_sc as plsc


_B, _S, _D = 4, 4096, 1024
_E = 128
_NW = 32
_RPW = (_B * _S) // _NW
_CH = 32
_NCH = _RPW // _CH
_SPW = _S // (_NW // _B)


def _sc_body(x_hbm, tab_hbm, out_hbm, o_v, sem_in, sem_out):
    wid = lax.axis_index("s") * 2 + lax.axis_index("c")
    b = wid // (_NW // _B)
    s0 = (wid % (_NW // _B)) * _SPW

    def in_copies(i, p):
        s = s0 + i * _CH
        return (
            pltpu.make_async_copy(
                x_hbm.at[b, pl.ds(s, _CH), :],
                o_v.at[p, :, pl.ds(0, _D)], sem_in.at[p]),
            pltpu.make_async_copy(
                tab_hbm.at[pl.ds(s, _CH), :],
                o_v.at[p, :, pl.ds(_D, _E)], sem_in.at[p]),
        )

    def out_copy(i, p):
        s = s0 + i * _CH
        return pltpu.make_async_copy(
            o_v.at[p], out_hbm.at[b, pl.ds(s, _CH), :], sem_out.at[p])

    for cp in in_copies(0, 0) + in_copies(1, 1):
        cp.start()
    for i in range(_NCH):
        p = i & 1
        for cp in in_copies(i, p):
            cp.wait()
        out_cp = out_copy(i, p)
        out_cp.start()
        if i + 2 < _NCH:
            out_cp.wait()
            for cp in in_copies(i + 2, p):
                cp.start()
    for i in (_NCH - 2, _NCH - 1):
        out_copy(i, i & 1).wait()


def kernel(x, embed_table):
    b, s, d = x.shape
    e = embed_table.shape[1]
    mesh = plsc.VectorSubcoreMesh(core_axis_name="c", subcore_axis_name="s")
    k = functools.partial(
        pl.kernel,
        mesh=mesh,
        out_type=jax.ShapeDtypeStruct((b, s, d + e), x.dtype),
        scratch_types=[
            pltpu.VMEM((2, _CH, _D + _E), jnp.float32),
            pltpu.SemaphoreType.DMA((2,)),
            pltpu.SemaphoreType.DMA((2,)),
        ],
    )(_sc_body)
    return k(x, embed_table)


# speedup vs baseline: 1.1384x; 1.1384x over previous
import functools

import jax
import jax.numpy as jnp
from jax import lax
from jax.experimental import pallas as pl
from jax.experimental.pallas import tpu as pltpu
from jax.experimental.pallas import tpu_sc as plsc


_B, _S, _D = 4, 4096, 1024
_E = 128
_NW = 32
_RPW = _S // _NW
_SB = 2048


def _sc_lookup(tab_hbm, pos_hbm, t_v, sem_in, sem_out):
    wid = lax.axis_index("s") * 2 + lax.axis_index("c")
    s0 = wid * _RPW
    cin = pltpu.make_async_copy(tab_hbm.at[pl.ds(s0, _RPW), :], t_v, sem_in)
    cin.start()
    cin.wait()
    cout = pltpu.make_async_copy(t_v, pos_hbm.at[pl.ds(s0, _RPW), :], sem_out)
    cout.start()
    cout.wait()


def _tc_concat(x_ref, pos_ref, out_ref):
    out_ref[:, :, :_D] = x_ref[...]
    out_ref[:, :, _D:] = pos_ref[...][None, :, :]


def kernel(x, embed_table):
    b, s, d = x.shape
    e = embed_table.shape[1]
    mesh = plsc.VectorSubcoreMesh(core_axis_name="c", subcore_axis_name="s")
    sc_lookup = functools.partial(
        pl.kernel,
        mesh=mesh,
        out_type=jax.ShapeDtypeStruct((s, e), x.dtype),
        scratch_types=[
            pltpu.VMEM((_RPW, _E), jnp.float32),
            pltpu.SemaphoreType.DMA,
            pltpu.SemaphoreType.DMA,
        ],
    )(_sc_lookup)
    pos_slab = sc_lookup(embed_table)

    return pl.pallas_call(
        _tc_concat,
        grid=(s // _SB, b),
        in_specs=[
            pl.BlockSpec((1, _SB, d), lambda i, j: (j, i, 0)),
            pl.BlockSpec((_SB, e), lambda i, j: (i, 0)),
        ],
        out_specs=pl.BlockSpec((1, _SB, d + e), lambda i, j: (j, i, 0)),
        out_shape=jax.ShapeDtypeStruct((b, s, d + e), x.dtype),
        compiler_params=pltpu.CompilerParams(
            dimension_semantics=("parallel", "parallel"),
        ),
    )(x, pos_slab)


# speedup vs baseline: 1.6658x; 1.4633x over previous
import jax
import jax.numpy as jnp
from jax.experimental import pallas as pl
from jax.experimental.pallas import tpu as pltpu


_B, _S, _D = 4, 4096, 1024
_E = 128
_SB = 2048


def _concat_kernel(x_ref, tab_ref, out_ref):
    out_ref[:, :, :_D] = x_ref[...]
    out_ref[:, :, _D:] = tab_ref[...][None, :, :]


def kernel(x, embed_table):
    b, s, d = x.shape
    e = embed_table.shape[1]
    grid = (s // _SB, b)
    return pl.pallas_call(
        _concat_kernel,
        grid=grid,
        in_specs=[
            pl.BlockSpec((1, _SB, d), lambda i, j: (j, i, 0)),
            pl.BlockSpec((_SB, e), lambda i, j: (i, 0)),
        ],
        out_specs=pl.BlockSpec((1, _SB, d + e), lambda i, j: (j, i, 0)),
        out_shape=jax.ShapeDtypeStruct((b, s, d + e), x.dtype),
        compiler_params=pltpu.CompilerParams(
            dimension_semantics=("parallel", "parallel"),
        ),
    )(x, embed_table)
